# baseline (speedup 1.0000x reference)
import functools

import jax
import jax.numpy as jnp
from jax import lax
from jax.experimental import pallas as pl
from jax.experimental.pallas import tpu as pltpu
from jax.experimental.pallas import tpu_sc as plsc

N = 10000
E = 320000
D = 128
R = 20
NB = 10
EPS = 1e-12

NPAD = 10240
NSEG = NPAD * R
NSEGH = NSEG // 2
ACCR = NSEGH + 128
NCHUNK = 16
W8 = 8

NTILE = 16
EPT = 20480
EPAD = EPT * NTILE
CH = 2048
NCH = EPT // CH
NB128 = CH // 128
STRIPE = NSEGH // NTILE

NBLK = 256
NGRID = NPAD // NBLK


def _sc_embed_body(xt, syn, pos, sense, lem, hraw, idxl, trows, hacc, sem):
  cid = lax.axis_index("c")
  sid = lax.axis_index("s")
  wid = sid * 2 + cid
  n0 = wid * 320
  for c in range(4):
    pltpu.sync_copy(xt.at[pl.ds(c * NPAD + n0, 320)],
                    idxl.at[pl.ds(c * 320, 320)])

  tables = (syn, pos, sense, lem)
  for b, bsz in ((0, 128), (128, 128), (256, 64)):
    pltpu.async_copy(tables[0].at[idxl.at[pl.ds(b, bsz)]],
                     hacc.at[pl.ds(b, bsz)], sem).wait()
    for t in (1, 2, 3):
      pltpu.async_copy(tables[t].at[idxl.at[pl.ds(t * 320 + b, bsz)]],
                       trows.at[pl.ds(0, bsz)], sem).wait()

      def add_loop(k, _):
        r = k // 8
        off = (k % 8) * 16
        plsc.addupdate(hacc.at[b + r, pl.ds(off, 16)],
                       trows[r, pl.ds(off, 16)])
        return _
      lax.fori_loop(0, bsz * 8, add_loop, 0)
  pltpu.sync_copy(hacc, hraw.at[pl.ds(n0, 320)])


def _sc_edges_body(srcp, dstp, relp, h2, ones2d, zrows, s_out, cnt_out,
                   ebuf, gidx2d, sidx2d, grows, zbuf, onesv, acc, gsem, ssem):
  cid = lax.axis_index("c")
  sid = lax.axis_index("s")
  r0 = sid * STRIPE
  seg0 = cid * NSEGH

  pltpu.sync_copy(ones2d, onesv)
  pltpu.sync_copy(zrows, zbuf)

  def zero_acc():
    for j in range(3):
      pltpu.sync_copy(zbuf, acc.at[pl.ds(r0 + j * CH, CH)])
    pltpu.sync_copy(zbuf.at[pl.ds(0, STRIPE - 3 * CH)],
                    acc.at[pl.ds(r0 + 3 * CH, STRIPE - 3 * CH)])

  def load_chunk(base, with_gather, cc):
    pltpu.sync_copy(srcp.at[pl.ds(base, CH)], ebuf.at[pl.ds(0, CH)])
    pltpu.sync_copy(dstp.at[pl.ds(base, CH)], ebuf.at[pl.ds(CH, CH)])
    pltpu.sync_copy(relp.at[pl.ds(base, CH)], ebuf.at[pl.ds(2 * CH, CH)])

    def idx_loop(g, _):
      b = g // 8
      off = (g % 8) * 16
      d16 = ebuf[pl.ds(CH + g * 16, 16)]
      r16 = ebuf[pl.ds(2 * CH + g * 16, 16)]
      sl = d16 * R + r16 - seg0
      inb = jnp.logical_and(sl >= 0, sl < NSEGH)
      sidx2d[b, pl.ds(off, 16)] = jnp.where(inb, sl, NSEGH)
      if with_gather:
        s16 = ebuf[pl.ds(g * 16, 16)]
        gidx2d[b, pl.ds(off, 16)] = s16 * NCHUNK + cc
      return _
    lax.fori_loop(0, CH // 16, idx_loop, 0)

  def scatter_sums():
    for b in range(NB128):
      pltpu.async_copy(grows.at[b], acc.at[sidx2d.at[b]], ssem, add=True)
    for b in range(NB128):
      pltpu.make_async_copy(grows.at[b], acc.at[sidx2d.at[b]], ssem).wait()

  def dump(dst_ref):
    plsc.subcore_barrier()
    pltpu.sync_copy(acc.at[pl.ds(r0, STRIPE)],
                    dst_ref.at[pl.ds(seg0 + r0, STRIPE)])
    plsc.subcore_barrier()

  def one_pass(cc, _):
    zero_acc()
    plsc.subcore_barrier()

    def one_chunk(ch, __):
      base = sid * EPT + ch * CH
      load_chunk(base, True, cc)
      for b in range(NB128):
        pltpu.async_copy(h2.at[gidx2d.at[b]], grows.at[b], gsem)
      for b in range(NB128):
        pltpu.make_async_copy(h2.at[gidx2d.at[b]], grows.at[b], gsem).wait()
      scatter_sums()
      return __
    lax.fori_loop(0, NCH, one_chunk, 0)
    dump(s_out.at[cc])
    return _
  lax.fori_loop(0, NCHUNK, one_pass, 0)

  zero_acc()
  plsc.subcore_barrier()

  def cnt_chunk(ch, _):
    base = sid * EPT + ch * CH
    load_chunk(base, False, 0)
    for b in range(NB128):
      pltpu.async_copy(onesv, acc.at[sidx2d.at[b]], ssem, add=True)
    for b in range(NB128):
      pltpu.make_async_copy(onesv, acc.at[sidx2d.at[b]], ssem).wait()
    return _
  lax.fori_loop(0, NCH, cnt_chunk, 0)
  dump(cnt_out)


def _tc_prep_body(hraw_ref, gamma_ref, beta_ref, root_ref, bias_ref,
                  h_ref, hr_ref):
  xb = hraw_ref[...]
  mu = jnp.mean(xb, axis=-1, keepdims=True)
  var = jnp.mean(jnp.square(xb - mu), axis=-1, keepdims=True)
  h = (xb - mu) * lax.rsqrt(var + EPS) * gamma_ref[...] + beta_ref[...]
  h_ref[...] = h
  hr_ref[...] = jnp.dot(h, root_ref[...],
                        preferred_element_type=jnp.float32) + bias_ref[...]


def _tc_wperm_body(comp_ref, basis_ref, w_ref):
  w_ref[0] = jnp.dot(comp_ref[...], basis_ref[...],
                     preferred_element_type=jnp.float32)


def _tc_final_body(s_ref, cnt_ref, wperm_ref, hr_ref, out_ref):
  inv = 1.0 / jnp.maximum(cnt_ref[...], 1.0)
  acc = hr_ref[...]
  for cc in range(NCHUNK):
    acc = acc + jnp.dot(s_ref[cc] * inv, wperm_ref[cc],
                        preferred_element_type=jnp.float32)
  out_ref[...] = acc


def kernel(x, edge_index, e_id, edge_attrs, syn_emb, pos_emb, sense_emb,
           lem_emb, ln_gamma, ln_beta, comp, basis, root, bias):
  del e_id
  f32 = jnp.float32
  mesh = plsc.VectorSubcoreMesh(core_axis_name="c", subcore_axis_name="s",
                                num_cores=2, num_subcores=NTILE)

  xp = jnp.zeros((NPAD, 4), jnp.int32).at[:N].set(x.astype(jnp.int32))
  src = edge_index[0].astype(jnp.int32)
  dst = edge_index[1].astype(jnp.int32)
  rel = edge_attrs.astype(jnp.int32)
  pad = EPAD - E
  srcp = jnp.concatenate([src, jnp.zeros((pad,), jnp.int32)])
  dstp = jnp.concatenate([dst, jnp.full((pad,), NPAD, jnp.int32)])
  relp = jnp.concatenate([rel, jnp.zeros((pad,), jnp.int32)])
  ones2d = jnp.ones((128, W8), f32)
  zrows = jnp.zeros((CH, W8), f32)

  sc_embed = pl.kernel(
      _sc_embed_body,
      out_type=jax.ShapeDtypeStruct((NPAD, D), f32),
      mesh=mesh,
      scratch_types=[
          pltpu.VMEM((1280,), jnp.int32),
          pltpu.VMEM((128, D), f32),
          pltpu.VMEM((320, D), f32),
          pltpu.SemaphoreType.DMA,
      ],
  )
  hraw = sc_embed(xp.T.reshape(-1), syn_emb, pos_emb, sense_emb, lem_emb)

  h, hr = pl.pallas_call(
      _tc_prep_body,
      grid=(NGRID,),
      in_specs=[
          pl.BlockSpec((NBLK, D), lambda i: (i, 0)),
          pl.BlockSpec((1, D), lambda i: (0, 0)),
          pl.BlockSpec((1, D), lambda i: (0, 0)),
          pl.BlockSpec((D, D), lambda i: (0, 0)),
          pl.BlockSpec((1, D), lambda i: (0, 0)),
      ],
      out_specs=[
          pl.BlockSpec((NBLK, D), lambda i: (i, 0)),
          pl.BlockSpec((NBLK, D), lambda i: (i, 0)),
      ],
      out_shape=[
          jax.ShapeDtypeStruct((NPAD, D), f32),
          jax.ShapeDtypeStruct((NPAD, D), f32),
      ],
  )(hraw, ln_gamma.reshape(1, D), ln_beta.reshape(1, D), root,
    bias.reshape(1, D))

  wflat = pl.pallas_call(
      _tc_wperm_body,
      grid=(NCHUNK,),
      in_specs=[
          pl.BlockSpec((R, NB), lambda i: (0, 0)),
          pl.BlockSpec((NB, 8 * D), lambda i: (0, i)),
      ],
      out_specs=pl.BlockSpec((1, R, 8 * D), lambda i: (i, 0, 0)),
      out_shape=jax.ShapeDtypeStruct((NCHUNK, R, 8 * D), f32),
  )(comp, basis.reshape(NB, D * D))
  wperm = wflat.reshape(NCHUNK, R * 8, D)

  sc_edges = pl.kernel(
      _sc_edges_body,
      compiler_params=pltpu.CompilerParams(use_tc_tiling_on_sc=False),
      out_type=(
          jax.ShapeDtypeStruct((NCHUNK, NSEG, W8), f32),
          jax.ShapeDtypeStruct((NSEG, W8), f32),
      ),
      mesh=mesh,
      scratch_types=[
          pltpu.VMEM((3 * CH,), jnp.int32),
          pltpu.VMEM((NB128, 128), jnp.int32),
          pltpu.VMEM((NB128, 128), jnp.int32),
          pltpu.VMEM((NB128, 128, W8), f32),
          pltpu.VMEM((CH, W8), f32),
          pltpu.VMEM((128, W8), f32),
          pltpu.VMEM_SHARED((ACCR, W8), f32),
          pltpu.SemaphoreType.DMA,
          pltpu.SemaphoreType.DMA,
      ],
  )
  s_sum, cnt = sc_edges(srcp, dstp, relp, h.reshape(NPAD * NCHUNK, W8),
                        ones2d, zrows)

  out = pl.pallas_call(
      _tc_final_body,
      grid=(NGRID,),
      in_specs=[
          pl.BlockSpec((NCHUNK, NBLK, R * W8), lambda i: (0, i, 0)),
          pl.BlockSpec((NBLK, R * W8), lambda i: (i, 0)),
          pl.BlockSpec((NCHUNK, R * 8, D), lambda i: (0, 0, 0)),
          pl.BlockSpec((NBLK, D), lambda i: (i, 0)),
      ],
      out_specs=pl.BlockSpec((NBLK, D), lambda i: (i, 0)),
      out_shape=jax.ShapeDtypeStruct((NPAD, D), f32),
  )(s_sum.reshape(NCHUNK, NPAD, R * W8), cnt.reshape(NPAD, R * W8),
    wperm, hr)

  return out[:N]


# speedup vs baseline: 2.3448x; 2.3448x over previous
import jax
import jax.numpy as jnp
from jax import lax
from jax.experimental import pallas as pl
from jax.experimental.pallas import tpu as pltpu
from jax.experimental.pallas import tpu_sc as plsc

N = 10000
E = 320000
D = 128
R = 20
NB = 10
EPS = 1e-12

NPAD = 10240
NSEG = NPAD * R
DW = 136
DBLK = 320
NBLKD = NPAD // DBLK
BSEG = DBLK * R
ACCR = BSEG + 8

NTILE = 16
EPT = 20480
EPAD = EPT * NTILE
ECH = 2048
NCH = EPT // ECH
LPAD = EPT + 128

NBLK = 256
NGRID = NPAD // NBLK


def _sc_embed_body(xt, syn, pos, sense, lem, hraw, idxl, trows, hacc, sem):
  cid = lax.axis_index("c")
  sid = lax.axis_index("s")
  wid = sid * 2 + cid
  n0 = wid * 320
  for c in range(4):
    pltpu.sync_copy(xt.at[pl.ds(c * NPAD + n0, 320)],
                    idxl.at[pl.ds(c * 320, 320)])

  tables = (syn, pos, sense, lem)
  for b, bsz in ((0, 128), (128, 128), (256, 64)):
    pltpu.async_copy(tables[0].at[idxl.at[pl.ds(b, bsz)]],
                     hacc.at[pl.ds(b, bsz)], sem).wait()
    for t in (1, 2, 3):
      pltpu.async_copy(tables[t].at[idxl.at[pl.ds(t * 320 + b, bsz)]],
                       trows.at[pl.ds(0, bsz)], sem).wait()

      def add_loop(k, _):
        r = k // 8
        off = (k % 8) * 16
        plsc.addupdate(hacc.at[b + r, pl.ds(off, 16)],
                       trows[r, pl.ds(off, 16)])
        return _
      lax.fori_loop(0, bsz * 8, add_loop, 0)
  pltpu.sync_copy(hacc, hraw.at[pl.ds(n0, 320)])


def _sc_edges_body(srcp, dstp, relp, hp, zrows, s_out,
                   ebuf, clist, sbatch, gbatch, grows, zbuf,
                   acc, gsem, ssem, esem):
  cid = lax.axis_index("c")
  sid = lax.axis_index("s")
  e0 = sid * EPT
  r0 = sid * (BSEG // NTILE)

  pltpu.sync_copy(zrows, zbuf)

  def fetch_edges(ch, par):
    base = e0 + ch * ECH
    o = par * 3 * ECH
    pltpu.async_copy(srcp.at[pl.ds(base, ECH)],
                     ebuf.at[pl.ds(o, ECH)], esem)
    pltpu.async_copy(dstp.at[pl.ds(base, ECH)],
                     ebuf.at[pl.ds(o + ECH, ECH)], esem)
    pltpu.async_copy(relp.at[pl.ds(base, ECH)],
                     ebuf.at[pl.ds(o + 2 * ECH, ECH)], esem)

  def wait_edges(par):
    o = par * 3 * ECH
    for j in range(3):
      pltpu.make_async_copy(srcp.at[pl.ds(0, ECH)],
                            ebuf.at[pl.ds(o + j * ECH, ECH)], esem).wait()

  def one_pass(p, _):
    blk = cid * (NBLKD // 2) + p
    lo = blk * BSEG

    for j in range(BSEG // NTILE // 40):
      pltpu.sync_copy(zbuf, acc.at[pl.ds(r0 + j * 40, 40)])
    plsc.subcore_barrier()

    fetch_edges(0, 0)

    def one_chunk(ch, wp):
      par = ch % 2
      wait_edges(par)

      @pl.when(ch + 1 < NCH)
      def _prefetch():
        fetch_edges(ch + 1, 1 - par)
      o = par * 3 * ECH

      def select(g, wpi):
        s16 = ebuf[pl.ds(o + g * 16, 16)]
        d16 = ebuf[pl.ds(o + ECH + g * 16, 16)]
        r16 = ebuf[pl.ds(o + 2 * ECH + g * 16, 16)]
        seg = d16 * R + r16
        m = jnp.logical_and(seg >= lo, seg < lo + BSEG)
        pc = plsc.cumsum(jnp.where(m, 1, 0))
        plsc.store_scatter(clist, [wpi + pc - 1],
                           lax.shift_left(seg, 14) | s16, mask=m)
        return wpi + jnp.max(pc)
      return lax.fori_loop(0, ECH // 16, select, wp)
    nsel = lax.fori_loop(0, NCH, one_chunk, 0)

    pv = jnp.full((16,), (lo + BSEG) * 16384, jnp.int32)
    for k in range(8):
      clist[pl.ds(nsel + k * 16, 16)] = pv
    nbat = (nsel + 127) // 128

    def one_batch(j, _):
      for k in range(8):
        p16 = clist[pl.ds(j * 128 + k * 16, 16)]
        sbatch[0, pl.ds(k * 16, 16)] = (
            lax.shift_right_logical(p16, 14) - lo)
        gbatch[pl.ds(k * 16, 16)] = p16 & 16383
      pltpu.async_copy(hp.at[gbatch], grows.at[0], gsem).wait()
      pltpu.async_copy(grows.at[0], acc.at[sbatch.at[0]], ssem, add=True).wait()
      return _
    lax.fori_loop(0, nbat, one_batch, 0)

    plsc.subcore_barrier()
    pltpu.sync_copy(acc.at[pl.ds(r0, BSEG // NTILE)],
                    s_out.at[pl.ds(lo + r0, BSEG // NTILE)])
    plsc.subcore_barrier()
    return _
  lax.fori_loop(0, NBLKD // 2, one_pass, 0)


def _tc_prep_body(hraw_ref, gamma_ref, beta_ref, root_ref, bias_ref,
                  h_ref, hr_ref):
  xb = hraw_ref[...]
  mu = jnp.mean(xb, axis=-1, keepdims=True)
  var = jnp.mean(jnp.square(xb - mu), axis=-1, keepdims=True)
  h = (xb - mu) * lax.rsqrt(var + EPS) * gamma_ref[...] + beta_ref[...]
  h_ref[...] = h
  hr_ref[...] = jnp.dot(h, root_ref[...],
                        preferred_element_type=jnp.float32) + bias_ref[...]


def _tc_wperm_body(comp_ref, basis_ref, w_ref):
  w_ref[...] = jnp.dot(comp_ref[...], basis_ref[...],
                       preferred_element_type=jnp.float32)


def _tc_final_body(s_ref, w_ref, hr_ref, out_ref):
  acc = hr_ref[...]
  for r in range(R):
    sl = s_ref[:, r * DW:r * DW + D]
    cnt = s_ref[:, r * DW + D:r * DW + D + 1]
    acc = acc + jnp.dot(sl * (1.0 / jnp.maximum(cnt, 1.0)), w_ref[r],
                        preferred_element_type=jnp.float32)
  out_ref[...] = acc


def kernel(x, edge_index, e_id, edge_attrs, syn_emb, pos_emb, sense_emb,
           lem_emb, ln_gamma, ln_beta, comp, basis, root, bias):
  del e_id
  f32 = jnp.float32
  mesh = plsc.VectorSubcoreMesh(core_axis_name="c", subcore_axis_name="s",
                                num_cores=2, num_subcores=NTILE)

  xp = jnp.zeros((NPAD, 4), jnp.int32).at[:N].set(x.astype(jnp.int32))
  src = edge_index[0].astype(jnp.int32)
  dst = edge_index[1].astype(jnp.int32)
  rel = edge_attrs.astype(jnp.int32)
  pad = EPAD - E
  srcp = jnp.concatenate([src, jnp.zeros((pad,), jnp.int32)])
  dstp = jnp.concatenate([dst, jnp.full((pad,), NPAD, jnp.int32)])
  relp = jnp.concatenate([rel, jnp.zeros((pad,), jnp.int32)])
  zrows = jnp.zeros((40, DW), f32)

  sc_embed = pl.kernel(
      _sc_embed_body,
      out_type=jax.ShapeDtypeStruct((NPAD, D), f32),
      mesh=mesh,
      scratch_types=[
          pltpu.VMEM((1280,), jnp.int32),
          pltpu.VMEM((128, D), f32),
          pltpu.VMEM((320, D), f32),
          pltpu.SemaphoreType.DMA,
      ],
  )
  hraw = sc_embed(xp.T.reshape(-1), syn_emb, pos_emb, sense_emb, lem_emb)

  h, hr = pl.pallas_call(
      _tc_prep_body,
      grid=(NGRID,),
      in_specs=[
          pl.BlockSpec((NBLK, D), lambda i: (i, 0)),
          pl.BlockSpec((1, D), lambda i: (0, 0)),
          pl.BlockSpec((1, D), lambda i: (0, 0)),
          pl.BlockSpec((D, D), lambda i: (0, 0)),
          pl.BlockSpec((1, D), lambda i: (0, 0)),
      ],
      out_specs=[
          pl.BlockSpec((NBLK, D), lambda i: (i, 0)),
          pl.BlockSpec((NBLK, D), lambda i: (i, 0)),
      ],
      out_shape=[
          jax.ShapeDtypeStruct((NPAD, D), f32),
          jax.ShapeDtypeStruct((NPAD, D), f32),
      ],
  )(hraw, ln_gamma.reshape(1, D), ln_beta.reshape(1, D), root,
    bias.reshape(1, D))

  hp = jnp.concatenate(
      [h, jnp.ones((NPAD, 1), f32), jnp.zeros((NPAD, DW - D - 1), f32)],
      axis=1)

  wflat = pl.pallas_call(
      _tc_wperm_body,
      grid=(1,),
      in_specs=[
          pl.BlockSpec((R, NB), lambda i: (0, 0)),
          pl.BlockSpec((NB, D * D), lambda i: (0, 0)),
      ],
      out_specs=pl.BlockSpec((R, D * D), lambda i: (0, 0)),
      out_shape=jax.ShapeDtypeStruct((R, D * D), f32),
  )(comp, basis.reshape(NB, D * D))
  weight = wflat.reshape(R, D, D)

  sc_edges = pl.kernel(
      _sc_edges_body,
      compiler_params=pltpu.CompilerParams(use_tc_tiling_on_sc=False,
                                           needs_layout_passes=False),
      out_type=jax.ShapeDtypeStruct((NSEG, DW), f32),
      mesh=mesh,
      scratch_types=[
          pltpu.VMEM((2 * 3 * ECH,), jnp.int32),
          pltpu.VMEM((LPAD,), jnp.int32),
          pltpu.VMEM((2, 128), jnp.int32),
          pltpu.VMEM((128,), jnp.int32),
          pltpu.VMEM((2, 128, DW), f32),
          pltpu.VMEM((40, DW), f32),
          pltpu.VMEM_SHARED((ACCR, DW), f32),
          pltpu.SemaphoreType.DMA,
          pltpu.SemaphoreType.DMA,
          pltpu.SemaphoreType.DMA,
      ],
  )
  s_sum = sc_edges(srcp, dstp, relp, hp, zrows)

  out = pl.pallas_call(
      _tc_final_body,
      grid=(NGRID,),
      in_specs=[
          pl.BlockSpec((NBLK, R * DW), lambda i: (i, 0)),
          pl.BlockSpec((R, D, D), lambda i: (0, 0, 0)),
          pl.BlockSpec((NBLK, D), lambda i: (i, 0)),
      ],
      out_specs=pl.BlockSpec((NBLK, D), lambda i: (i, 0)),
      out_shape=jax.ShapeDtypeStruct((NPAD, D), f32),
  )(s_sum.reshape(NPAD, R * DW), weight, hr)

  return out[:N]
